# baseline (speedup 1.0000x reference)
import jax
import jax.numpy as jnp
from jax.experimental import pallas as pl
from jax.experimental.pallas import tpu as pltpu

_BN = 16384


def _sims_body(f_ref, m_ref, o_ref):
    f = f_ref[0]
    norm = jnp.sqrt(jnp.sum(f * f, axis=1, keepdims=True))
    f = f / jnp.maximum(norm, 1e-12)
    m = m_ref[0]
    o_ref[0] = jax.lax.dot_general(
        f, m, (((1,), (1,)), ((), ())), preferred_element_type=jnp.float32
    )


def kernel(part_features, memory):
    k, b, d = part_features.shape
    _, n, _ = memory.shape
    bn = min(_BN, n)
    grid = (k, pl.cdiv(n, bn))
    return pl.pallas_call(
        _sims_body,
        grid=grid,
        in_specs=[
            pl.BlockSpec((1, b, d), lambda ki, ji: (ki, 0, 0)),
            pl.BlockSpec((1, bn, d), lambda ki, ji: (ki, ji, 0)),
        ],
        out_specs=pl.BlockSpec((1, b, bn), lambda ki, ji: (ki, 0, ji)),
        out_shape=jax.ShapeDtypeStruct((k, b, n), jnp.float32),
        compiler_params=pltpu.CompilerParams(
            dimension_semantics=("parallel", "parallel"),
        ),
    )(part_features, memory)


# speedup vs baseline: 1.0709x; 1.0709x over previous
import jax
import jax.numpy as jnp
from jax.experimental import pallas as pl
from jax.experimental.pallas import tpu as pltpu

_BN = 12544
_NBUF = 3


def _sims_body(f_ref, mem_hbm, o_ref, mbuf, sem):
    k = pl.program_id(0)
    j = pl.program_id(1)
    nj = pl.num_programs(1)
    total = pl.num_programs(0) * nj
    n_total = mem_hbm.shape[1]
    tail = n_total - (nj - 1) * _BN

    def _start(step):
        slot = jax.lax.rem(step, _NBUF)
        ks = jax.lax.div(step, nj)
        js = jax.lax.rem(step, nj)

        @pl.when(js < nj - 1)
        def _():
            pltpu.make_async_copy(
                mem_hbm.at[ks, pl.ds(js * _BN, _BN), :],
                mbuf.at[slot],
                sem.at[slot],
            ).start()

        @pl.when(js == nj - 1)
        def _():
            pltpu.make_async_copy(
                mem_hbm.at[ks, pl.ds((nj - 1) * _BN, tail), :],
                mbuf.at[slot, pl.ds(0, tail), :],
                sem.at[slot],
            ).start()

    i = k * nj + j

    @pl.when(i == 0)
    def _():
        for s in range(_NBUF):
            _start(s)

    slot = jax.lax.rem(i, _NBUF)

    @pl.when(j < nj - 1)
    def _():
        pltpu.make_async_copy(
            mem_hbm.at[k, pl.ds(j * _BN, _BN), :],
            mbuf.at[slot],
            sem.at[slot],
        ).wait()

    @pl.when(j == nj - 1)
    def _():
        pltpu.make_async_copy(
            mem_hbm.at[k, pl.ds((nj - 1) * _BN, tail), :],
            mbuf.at[slot, pl.ds(0, tail), :],
            sem.at[slot],
        ).wait()

    f = f_ref[0]
    norm = jnp.sqrt(jnp.sum(f * f, axis=1, keepdims=True))
    f = f / jnp.maximum(norm, 1e-12)
    o_ref[0] = jax.lax.dot_general(
        f, mbuf[slot], (((1,), (1,)), ((), ())), preferred_element_type=jnp.float32
    )

    @pl.when(i + _NBUF < total)
    def _():
        _start(i + _NBUF)


def kernel(part_features, memory):
    k, b, d = part_features.shape
    _, n, _ = memory.shape
    bn = min(_BN, n)
    grid = (k, pl.cdiv(n, bn))
    return pl.pallas_call(
        _sims_body,
        grid=grid,
        in_specs=[
            pl.BlockSpec((1, b, d), lambda ki, ji: (ki, 0, 0)),
            pl.BlockSpec(memory_space=pltpu.MemorySpace.HBM),
        ],
        out_specs=pl.BlockSpec((1, b, bn), lambda ki, ji: (ki, 0, ji)),
        out_shape=jax.ShapeDtypeStruct((k, b, n), jnp.float32),
        scratch_shapes=[
            pltpu.VMEM((_NBUF, bn, d), jnp.float32),
            pltpu.SemaphoreType.DMA((_NBUF,)),
        ],
    )(part_features, memory)


# speedup vs baseline: 1.0718x; 1.0008x over previous
import jax
import jax.numpy as jnp
from jax.experimental import pallas as pl
from jax.experimental.pallas import tpu as pltpu

_BN = 12544
_NBUF = 3


def _sims_body(f_ref, mem_hbm, o_ref, mbuf, sem):
    k = pl.program_id(0)
    j = pl.program_id(1)
    nj = pl.num_programs(1)
    total = pl.num_programs(0) * nj
    n_total = mem_hbm.shape[1]
    tail = n_total - (nj - 1) * _BN

    half = _BN // 2
    tail2 = tail - half

    def _start(step):
        slot = jax.lax.rem(step, _NBUF)
        ks = jax.lax.div(step, nj)
        js = jax.lax.rem(step, nj)

        @pl.when(js < nj - 1)
        def _():
            pltpu.make_async_copy(
                mem_hbm.at[ks, pl.ds(js * _BN, half), :],
                mbuf.at[slot, pl.ds(0, half), :],
                sem.at[slot, 0],
            ).start()
            pltpu.make_async_copy(
                mem_hbm.at[ks, pl.ds(js * _BN + half, half), :],
                mbuf.at[slot, pl.ds(half, half), :],
                sem.at[slot, 1],
            ).start()

        @pl.when(js == nj - 1)
        def _():
            pltpu.make_async_copy(
                mem_hbm.at[ks, pl.ds((nj - 1) * _BN, half), :],
                mbuf.at[slot, pl.ds(0, half), :],
                sem.at[slot, 0],
            ).start()
            pltpu.make_async_copy(
                mem_hbm.at[ks, pl.ds((nj - 1) * _BN + half, tail2), :],
                mbuf.at[slot, pl.ds(half, tail2), :],
                sem.at[slot, 1],
            ).start()

    i = k * nj + j

    @pl.when(i == 0)
    def _():
        for s in range(_NBUF):
            _start(s)

    slot = jax.lax.rem(i, _NBUF)

    @pl.when(j < nj - 1)
    def _():
        pltpu.make_async_copy(
            mem_hbm.at[k, pl.ds(j * _BN, half), :],
            mbuf.at[slot, pl.ds(0, half), :],
            sem.at[slot, 0],
        ).wait()
        pltpu.make_async_copy(
            mem_hbm.at[k, pl.ds(j * _BN + half, half), :],
            mbuf.at[slot, pl.ds(half, half), :],
            sem.at[slot, 1],
        ).wait()

    @pl.when(j == nj - 1)
    def _():
        pltpu.make_async_copy(
            mem_hbm.at[k, pl.ds((nj - 1) * _BN, half), :],
            mbuf.at[slot, pl.ds(0, half), :],
            sem.at[slot, 0],
        ).wait()
        pltpu.make_async_copy(
            mem_hbm.at[k, pl.ds((nj - 1) * _BN + half, tail2), :],
            mbuf.at[slot, pl.ds(half, tail2), :],
            sem.at[slot, 1],
        ).wait()

    f = f_ref[0]
    norm = jnp.sqrt(jnp.sum(f * f, axis=1, keepdims=True))
    f = f / jnp.maximum(norm, 1e-12)
    o_ref[0] = jax.lax.dot_general(
        f, mbuf[slot], (((1,), (1,)), ((), ())), preferred_element_type=jnp.float32
    )

    @pl.when(i + _NBUF < total)
    def _():
        _start(i + _NBUF)


def kernel(part_features, memory):
    k, b, d = part_features.shape
    _, n, _ = memory.shape
    bn = min(_BN, n)
    grid = (k, pl.cdiv(n, bn))
    return pl.pallas_call(
        _sims_body,
        grid=grid,
        in_specs=[
            pl.BlockSpec((1, b, d), lambda ki, ji: (ki, 0, 0)),
            pl.BlockSpec(memory_space=pltpu.MemorySpace.HBM),
        ],
        out_specs=pl.BlockSpec((1, b, bn), lambda ki, ji: (ki, 0, ji)),
        out_shape=jax.ShapeDtypeStruct((k, b, n), jnp.float32),
        scratch_shapes=[
            pltpu.VMEM((_NBUF, bn, d), jnp.float32),
            pltpu.SemaphoreType.DMA((_NBUF, 2)),
        ],
    )(part_features, memory)
